# baseline (speedup 1.0000x reference)
import functools

import jax
import jax.numpy as jnp
from jax import lax
from jax.experimental import pallas as pl
from jax.experimental.pallas import tpu as pltpu
from jax.experimental.pallas import tpu_sc as plsc

_B = 16384
_D = 64
_F = 64
_A = 10

_NC, _NS = 2, 16
_NW = _NC * _NS
_BPW = _B // _NW
_HPW = _BPW // 2


def _sc_gather(se, sa_g, obs, nobs, act):
    mesh = plsc.VectorSubcoreMesh(core_axis_name="c", subcore_axis_name="s")

    @functools.partial(
        pl.kernel,
        mesh=mesh,
        out_type=[
            jax.ShapeDtypeStruct((_B // 2, 2 * _D), jnp.float32),
            jax.ShapeDtypeStruct((_B // 2, 2 * _D), jnp.float32),
        ],
        scratch_types=[
            pltpu.VMEM((_BPW,), jnp.int32),
            pltpu.VMEM((_BPW,), jnp.int32),
            pltpu.VMEM((_BPW,), jnp.int32),
            pltpu.VMEM((_HPW, 2 * _D), jnp.float32),
            pltpu.VMEM((_HPW, 2 * _D), jnp.float32),
            pltpu.SemaphoreType.DMA,
        ],
    )
    def k(se_hbm, sa_hbm, obs_hbm, nobs_hbm, act_hbm,
          emb_out, sa_out,
          obs_v, nobs_v, act_v, emb_v, sa_v, sem):
        wid = lax.axis_index("s") * _NC + lax.axis_index("c")
        base = wid * _BPW
        pltpu.sync_copy(obs_hbm.at[pl.ds(base, _BPW)], obs_v)
        pltpu.sync_copy(nobs_hbm.at[pl.ds(base, _BPW)], nobs_v)
        pltpu.sync_copy(act_hbm.at[pl.ds(base, _BPW)], act_v)

        def make_fire(off):
            def fire(c, _):
                cb = c * 16
                src = off * _HPW + cb
                obs16 = obs_v[pl.ds(src, 16)]
                nobs16 = nobs_v[pl.ds(src, 16)]
                act16 = act_v[pl.ds(src, 16)]
                for j in range(16):
                    dst = (cb + j, pl.ds(off * _D, _D))
                    pltpu.async_copy(se_hbm.at[obs16[j]], emb_v.at[dst], sem)
                    pltpu.async_copy(sa_hbm.at[act16[j], nobs16[j]],
                                     sa_v.at[dst], sem)
                return 0
            return fire

        lax.fori_loop(0, _HPW // 16, make_fire(0), 0)
        lax.fori_loop(0, _HPW // 16, make_fire(1), 0)
        pltpu.make_async_copy(emb_out.at[pl.ds(0, _HPW)], emb_v, sem).wait()
        pltpu.make_async_copy(emb_out.at[pl.ds(0, _HPW)], sa_v, sem).wait()
        pltpu.sync_copy(emb_v, emb_out.at[pl.ds(wid * _HPW, _HPW)])
        pltpu.sync_copy(sa_v, sa_out.at[pl.ds(wid * _HPW, _HPW)])

    return k(se, sa_g, obs, nobs, act)


_COS_C = (
    0.9999999999999938, -19.73920880217503, 64.93939402216306,
    -85.45681717974715, 60.24464064338281, -26.42624548946228,
    7.903429882766466, -1.7137692085152525, 0.27980881692562937,
    -0.032045487143534404,
)
_INV_2PI = 0.15915494309189535


def _fast_cos(x):
    r = x * _INV_2PI
    r = r - jnp.round(r)
    u = r * r
    p = jnp.full_like(u, _COS_C[-1])
    for c in _COS_C[-2::-1]:
        p = p * u + c
    return p


def _tc_body(emb_ref, sa_ref, act_ref, om2_ref, sh2_ref, ae2_ref, aq2_ref,
             pol2_ref, out_ref):
    ae = ae2_ref[...]
    std = jnp.sqrt(jnp.maximum(1e-8, aq2_ref[...] - ae * ae))
    x = (emb_ref[...] - ae) / std
    proj = jnp.dot(x, om2_ref[...],
                   preferred_element_type=jnp.float32) * (1.0 / (_D ** 0.5))
    el = _fast_cos(proj + sh2_ref[...])
    logits = jnp.dot(el, pol2_ref[...],
                     preferred_element_type=jnp.float32)
    ne = _fast_cos(jnp.dot(sa_ref[...], om2_ref[...],
                           preferred_element_type=jnp.float32)
                   + sh2_ref[...])
    elne = el * ne
    acts = act_ref[...]
    for h in range(2):
        lg = logits[:, h * _A:(h + 1) * _A]
        m = jnp.max(lg, axis=1, keepdims=True)
        e = jnp.exp(lg - m)
        probs = e / jnp.sum(e, axis=1, keepdims=True)
        et = jnp.sum(elne[:, h * _D:(h + 1) * _D], axis=1,
                     keepdims=True) * ((2.0 / _F) ** 0.5)
        iota = lax.broadcasted_iota(jnp.int32, (_HPW, _A), 1)
        ap = jnp.sum(jnp.where(iota == acts[:, h:h + 1], probs, 0.0),
                     axis=1, keepdims=True)
        out_ref[pl.ds(h * _HPW, _HPW), :] = jnp.concatenate(
            [probs, ap, et], axis=1)


def _tc_dense(emb_p, sa_p, act2, omega2, shift2, ae2, aq2, policy2):
    return pl.pallas_call(
        _tc_body,
        grid=(_NW,),
        in_specs=[
            pl.BlockSpec((_HPW, 2 * _D), lambda i: (i, 0)),
            pl.BlockSpec((_HPW, 2 * _D), lambda i: (i, 0)),
            pl.BlockSpec((_HPW, 2), lambda i: (i, 0)),
            pl.BlockSpec((2 * _D, 2 * _F), lambda i: (0, 0)),
            pl.BlockSpec((1, 2 * _F), lambda i: (0, 0)),
            pl.BlockSpec((1, 2 * _D), lambda i: (0, 0)),
            pl.BlockSpec((1, 2 * _D), lambda i: (0, 0)),
            pl.BlockSpec((2 * _F, 2 * _A), lambda i: (0, 0)),
        ],
        out_specs=pl.BlockSpec((_BPW, _A + 2), lambda i: (i, 0)),
        out_shape=jax.ShapeDtypeStruct((_B, _A + 2), jnp.float32),
    )(emb_p, sa_p, act2, omega2, shift2, ae2, aq2, policy2)


def kernel(observation, action, next_observation, state_embedder,
           state_action_embedder, omega, shift, average_embed,
           average_square, embed_policy):
    sa_g = jnp.transpose(state_action_embedder, (1, 0, 2))
    emb_p, sa_p = _sc_gather(state_embedder, sa_g, observation,
                             next_observation, action)
    omt = omega.T
    zdd = jnp.zeros((_D, _F), jnp.float32)
    om2 = jnp.concatenate([jnp.concatenate([omt, zdd], axis=1),
                           jnp.concatenate([zdd, omt], axis=1)], axis=0)
    zfa = jnp.zeros((_F, _A), jnp.float32)
    pol2 = jnp.concatenate([jnp.concatenate([embed_policy, zfa], axis=1),
                            jnp.concatenate([zfa, embed_policy], axis=1)],
                           axis=0)
    act_p = action.reshape(_NW, 2, _HPW).transpose(0, 2, 1).reshape(_B // 2, 2)
    tile2 = lambda v: jnp.concatenate([v, v]).reshape(1, -1)
    return _tc_dense(
        emb_p, sa_p,
        act_p,
        om2,
        tile2(shift),
        tile2(average_embed),
        tile2(average_square),
        pol2,
    )


# speedup vs baseline: 1.0592x; 1.0592x over previous
import functools

import jax
import jax.numpy as jnp
from jax import lax
from jax.experimental import pallas as pl
from jax.experimental.pallas import tpu as pltpu
from jax.experimental.pallas import tpu_sc as plsc

_B = 16384
_D = 64
_F = 64
_A = 10

_NC, _NS = 2, 16
_NW = _NC * _NS
_BPW = _B // _NW
_HPW = _BPW // 2


def _sc_gather(se, sa_g, obs, nobs, act):
    mesh = plsc.VectorSubcoreMesh(core_axis_name="c", subcore_axis_name="s")

    @functools.partial(
        pl.kernel,
        mesh=mesh,
        out_type=[
            jax.ShapeDtypeStruct((_B // 2, 2 * _D), jnp.float32),
            jax.ShapeDtypeStruct((_B // 2, 2 * _D), jnp.float32),
        ],
        scratch_types=[
            pltpu.VMEM((_BPW,), jnp.int32),
            pltpu.VMEM((_BPW,), jnp.int32),
            pltpu.VMEM((_BPW,), jnp.int32),
            pltpu.VMEM((_HPW, 2 * _D), jnp.float32),
            pltpu.VMEM((_HPW, 2 * _D), jnp.float32),
            pltpu.SemaphoreType.DMA,
        ],
    )
    def k(se_hbm, sa_hbm, obs_hbm, nobs_hbm, act_hbm,
          emb_out, sa_out,
          obs_v, nobs_v, act_v, emb_v, sa_v, sem):
        wid = lax.axis_index("s") * _NC + lax.axis_index("c")
        base = wid * _BPW
        pltpu.sync_copy(obs_hbm.at[pl.ds(base, _BPW)], obs_v)
        pltpu.sync_copy(nobs_hbm.at[pl.ds(base, _BPW)], nobs_v)
        pltpu.sync_copy(act_hbm.at[pl.ds(base, _BPW)], act_v)

        def make_fire(off):
            def fire(c, _):
                cb = c * 16
                src = off * _HPW + cb
                obs16 = obs_v[pl.ds(src, 16)]
                nobs16 = nobs_v[pl.ds(src, 16)]
                act16 = act_v[pl.ds(src, 16)]
                for j in range(16):
                    dst = (cb + j, pl.ds(off * _D, _D))
                    pltpu.async_copy(se_hbm.at[obs16[j]], emb_v.at[dst], sem)
                    pltpu.async_copy(sa_hbm.at[act16[j], nobs16[j]],
                                     sa_v.at[dst], sem)
                return 0
            return fire

        lax.fori_loop(0, _HPW // 16, make_fire(0), 0)
        lax.fori_loop(0, _HPW // 16, make_fire(1), 0)
        pltpu.make_async_copy(emb_out.at[pl.ds(0, _HPW)], emb_v, sem).wait()
        pltpu.make_async_copy(emb_out.at[pl.ds(0, _HPW)], sa_v, sem).wait()
        pltpu.sync_copy(emb_v, emb_out.at[pl.ds(wid * _HPW, _HPW)])
        pltpu.sync_copy(sa_v, sa_out.at[pl.ds(wid * _HPW, _HPW)])

    return k(se, sa_g, obs, nobs, act)


_COS_C = (
    0.9999999999999938, -19.73920880217503, 64.93939402216306,
    -85.45681717974715, 60.24464064338281, -26.42624548946228,
    7.903429882766466, -1.7137692085152525, 0.27980881692562937,
    -0.032045487143534404,
)
_INV_2PI = 0.15915494309189535


def _fast_cos(x):
    r = x * _INV_2PI
    r = r - jnp.round(r)
    u = r * r
    p = jnp.full_like(u, _COS_C[-1])
    for c in _COS_C[-2::-1]:
        p = p * u + c
    return p


def _tc_body(emb_ref, sa_ref, act_ref, om_ref, sh_ref, ae_ref, aq_ref, pol_ref,
             out_ref):
    ep = emb_ref[...]
    sp = sa_ref[...]
    x = jnp.concatenate([ep[:, :_D], ep[:, _D:]], axis=0)
    sa = jnp.concatenate([sp[:, :_D], sp[:, _D:]], axis=0)
    ae = ae_ref[...]
    std = jnp.sqrt(jnp.maximum(1e-8, aq_ref[...] - ae * ae))
    x = (x - ae) / std
    om = om_ref[...]
    proj = lax.dot_general(x, om, (((1,), (1,)), ((), ())),
                           preferred_element_type=jnp.float32)
    proj = proj * (1.0 / (_D ** 0.5))
    el = _fast_cos(proj + sh_ref[...])
    logits = jnp.dot(el, pol_ref[...], preferred_element_type=jnp.float32)
    m = jnp.max(logits, axis=1, keepdims=True)
    e = jnp.exp(logits - m)
    probs = e / jnp.sum(e, axis=1, keepdims=True)
    ne = _fast_cos(lax.dot_general(sa, om, (((1,), (1,)), ((), ())),
                                   preferred_element_type=jnp.float32)
                   + sh_ref[...])
    et = jnp.sum(el * ne, axis=1, keepdims=True) * ((2.0 / _F) ** 0.5)
    iota = lax.broadcasted_iota(jnp.int32, (_BPW, _A), 1)
    ap = jnp.sum(jnp.where(iota == act_ref[...], probs, 0.0),
                 axis=1, keepdims=True)
    out_ref[...] = jnp.concatenate([probs, ap, et], axis=1)


def _tc_dense(emb_p, sa_p, act2, omega, shift2, ae2, aq2, policy):
    return pl.pallas_call(
        _tc_body,
        grid=(_NW,),
        in_specs=[
            pl.BlockSpec((_HPW, 2 * _D), lambda i: (i, 0)),
            pl.BlockSpec((_HPW, 2 * _D), lambda i: (i, 0)),
            pl.BlockSpec((_BPW, 1), lambda i: (i, 0)),
            pl.BlockSpec((_F, _D), lambda i: (0, 0)),
            pl.BlockSpec((1, _F), lambda i: (0, 0)),
            pl.BlockSpec((1, _D), lambda i: (0, 0)),
            pl.BlockSpec((1, _D), lambda i: (0, 0)),
            pl.BlockSpec((_F, _A), lambda i: (0, 0)),
        ],
        out_specs=pl.BlockSpec((_BPW, _A + 2), lambda i: (i, 0)),
        out_shape=jax.ShapeDtypeStruct((_B, _A + 2), jnp.float32),
    )(emb_p, sa_p, act2, omega, shift2, ae2, aq2, policy)


def kernel(observation, action, next_observation, state_embedder,
           state_action_embedder, omega, shift, average_embed,
           average_square, embed_policy):
    sa_g = jnp.transpose(state_action_embedder, (1, 0, 2))
    emb_p, sa_p = _sc_gather(state_embedder, sa_g, observation,
                             next_observation, action)
    return _tc_dense(
        emb_p, sa_p,
        action.reshape(_B, 1),
        omega,
        shift.reshape(1, _F),
        average_embed.reshape(1, _D),
        average_square.reshape(1, _D),
        embed_policy,
    )
